# baseline (device time: 19759 ns/iter reference)
import jax
import jax.numpy as jnp
from jax import lax
from jax.experimental import pallas as pl
from jax.experimental.pallas import tpu as pltpu

C = 4


def kernel(dy, W):
    m, f = dy.shape
    d, _ = W.shape
    half = m // 2
    ck = half // C

    def body(dy_ref, w_ref, out_ref, dyv_ref, wv_ref, acc_ref, recv_y_ref,
             load_sems, send_sems_y, recv_sems_y, send_sems_x, recv_sems_x):
        my_x = lax.axis_index("x")
        my_y = lax.axis_index("y")
        my_z = lax.axis_index("z")
        nbr_y = (my_x, 1 - my_y, my_z)
        nbr_x = (1 - my_x, my_y, my_z)
        base = my_x * half

        w_load = pltpu.make_async_copy(w_ref, wv_ref, load_sems.at[0])
        w_load.start()
        dy_load = pltpu.make_async_copy(
            dy_ref.at[pl.ds(base, half)], dyv_ref, load_sems.at[1]
        )
        dy_load.start()

        barrier_sem = pltpu.get_barrier_semaphore()
        for nbr in (nbr_y, nbr_x):
            pl.semaphore_signal(
                barrier_sem, inc=1,
                device_id=nbr, device_id_type=pl.DeviceIdType.MESH,
            )
        pl.semaphore_wait(barrier_sem, 2)
        dy_load.wait()
        w_load.wait()

        rdmas_y = []
        for i in range(C):
            sl = pl.ds(i * ck, ck)
            acc_ref[sl, :] = lax.dot_general(
                dyv_ref[sl, :], wv_ref[:, :],
                dimension_numbers=(((1,), (1,)), ((), ())),
                preferred_element_type=jnp.float32,
            )
            rdma = pltpu.make_async_remote_copy(
                src_ref=acc_ref.at[sl],
                dst_ref=recv_y_ref.at[sl],
                send_sem=send_sems_y.at[i],
                recv_sem=recv_sems_y.at[i],
                device_id=nbr_y,
                device_id_type=pl.DeviceIdType.MESH,
            )
            rdma.start()
            rdmas_y.append(rdma)

        rdmas_x = []
        for i in range(C):
            sl = pl.ds(i * ck, ck)
            out_sl = pl.ds(base + i * ck, ck)
            rdmas_y[i].wait_recv()
            out_ref[out_sl, :] = acc_ref[sl, :] + recv_y_ref[sl, :]
            rdma = pltpu.make_async_remote_copy(
                src_ref=out_ref.at[out_sl],
                dst_ref=out_ref.at[out_sl],
                send_sem=send_sems_x.at[i],
                recv_sem=recv_sems_x.at[i],
                device_id=nbr_x,
                device_id_type=pl.DeviceIdType.MESH,
            )
            rdma.start()
            rdmas_x.append(rdma)

        for i in range(C):
            rdmas_y[i].wait_send()
            rdmas_x[i].wait()

    return pl.pallas_call(
        body,
        out_shape=jax.ShapeDtypeStruct((m, d), jnp.float32),
        in_specs=[
            pl.BlockSpec(memory_space=pl.ANY),
            pl.BlockSpec(memory_space=pl.ANY),
        ],
        out_specs=pl.BlockSpec(memory_space=pltpu.VMEM),
        scratch_shapes=[
            pltpu.VMEM((half, f), jnp.float32),
            pltpu.VMEM((d, f), jnp.float32),
            pltpu.VMEM((half, d), jnp.float32),
            pltpu.VMEM((half, d), jnp.float32),
            pltpu.SemaphoreType.DMA((2,)),
            pltpu.SemaphoreType.DMA((C,)),
            pltpu.SemaphoreType.DMA((C,)),
            pltpu.SemaphoreType.DMA((C,)),
            pltpu.SemaphoreType.DMA((C,)),
        ],
        compiler_params=pltpu.CompilerParams(collective_id=0),
    )(dy, W)


# device time: 15902 ns/iter; 1.2425x vs baseline; 1.2425x over previous
import jax
import jax.numpy as jnp
from jax import lax
from jax.experimental import pallas as pl
from jax.experimental.pallas import tpu as pltpu

C = 4


def kernel(dy, W):
    m, f = dy.shape
    d, _ = W.shape
    half = m // 2
    ck = half // C

    def body(dy_ref, w_ref, out_ref, accb_ref, recv_y_ref, fwd_ref, recv_x_ref,
             send_sems_y, recv_sems_y, send_sems_x, recv_sems_x):
        my_x = lax.axis_index("x")
        my_y = lax.axis_index("y")
        my_z = lax.axis_index("z")
        nbr_y = (my_x, 1 - my_y, my_z)
        nbr_x = (1 - my_x, my_y, my_z)
        base = my_x * half
        obase = (1 - my_x) * half

        barrier_sem = pltpu.get_barrier_semaphore()
        for nbr in (nbr_y, nbr_x):
            pl.semaphore_signal(
                barrier_sem, inc=1,
                device_id=nbr, device_id_type=pl.DeviceIdType.MESH,
            )

        rdmas_y = []
        for i in range(C):
            sl = pl.ds(i * ck, ck)
            accb_ref[sl, :] = lax.dot_general(
                dy_ref[pl.ds(base + i * ck, ck), :], w_ref[:, :],
                dimension_numbers=(((1,), (1,)), ((), ())),
                preferred_element_type=jnp.float32,
            ).astype(jnp.bfloat16)
            if i == 0:
                pl.semaphore_wait(barrier_sem, 2)
            rdma = pltpu.make_async_remote_copy(
                src_ref=accb_ref.at[sl],
                dst_ref=recv_y_ref.at[sl],
                send_sem=send_sems_y.at[i],
                recv_sem=recv_sems_y.at[i],
                device_id=nbr_y,
                device_id_type=pl.DeviceIdType.MESH,
            )
            rdma.start()
            rdmas_y.append(rdma)

        rdmas_x = []
        for i in range(C):
            sl = pl.ds(i * ck, ck)
            rdmas_y[i].wait_recv()
            s = (accb_ref[sl, :].astype(jnp.float32)
                 + recv_y_ref[sl, :].astype(jnp.float32))
            out_ref[pl.ds(base + i * ck, ck), :] = s
            fwd_ref[sl, :] = s.astype(jnp.bfloat16)
            rdma = pltpu.make_async_remote_copy(
                src_ref=fwd_ref.at[sl],
                dst_ref=recv_x_ref.at[sl],
                send_sem=send_sems_x.at[i],
                recv_sem=recv_sems_x.at[i],
                device_id=nbr_x,
                device_id_type=pl.DeviceIdType.MESH,
            )
            rdma.start()
            rdmas_x.append(rdma)

        for i in range(C):
            sl = pl.ds(i * ck, ck)
            rdmas_x[i].wait_recv()
            out_ref[pl.ds(obase + i * ck, ck), :] = (
                recv_x_ref[sl, :].astype(jnp.float32)
            )
        for i in range(C):
            rdmas_y[i].wait_send()
            rdmas_x[i].wait_send()

    return pl.pallas_call(
        body,
        out_shape=jax.ShapeDtypeStruct((m, d), jnp.float32),
        in_specs=[
            pl.BlockSpec(memory_space=pltpu.VMEM),
            pl.BlockSpec(memory_space=pltpu.VMEM),
        ],
        out_specs=pl.BlockSpec(memory_space=pltpu.VMEM),
        scratch_shapes=[
            pltpu.VMEM((half, d), jnp.bfloat16),
            pltpu.VMEM((half, d), jnp.bfloat16),
            pltpu.VMEM((half, d), jnp.bfloat16),
            pltpu.VMEM((half, d), jnp.bfloat16),
            pltpu.SemaphoreType.DMA((C,)),
            pltpu.SemaphoreType.DMA((C,)),
            pltpu.SemaphoreType.DMA((C,)),
            pltpu.SemaphoreType.DMA((C,)),
        ],
        compiler_params=pltpu.CompilerParams(collective_id=0),
    )(dy, W)


# device time: 15335 ns/iter; 1.2885x vs baseline; 1.0370x over previous
import jax
import jax.numpy as jnp
from jax import lax
from jax.experimental import pallas as pl
from jax.experimental.pallas import tpu as pltpu

C = 4


def kernel(dy, W):
    m, f = dy.shape
    d, _ = W.shape
    ck = m // C

    def body(dy_ref, w_ref, out_ref, accb_ref, recv_ref,
             send_sems, recv_sems):
        my_x = lax.axis_index("x")
        my_y = lax.axis_index("y")
        my_z = lax.axis_index("z")
        nbr_y = (my_x, 1 - my_y, my_z)

        barrier_sem = pltpu.get_barrier_semaphore()
        pl.semaphore_signal(
            barrier_sem, inc=1,
            device_id=nbr_y, device_id_type=pl.DeviceIdType.MESH,
        )

        rdmas = []
        for i in range(C):
            sl = pl.ds(i * ck, ck)
            accb_ref[sl, :] = lax.dot_general(
                dy_ref[sl, :], w_ref[:, :],
                dimension_numbers=(((1,), (1,)), ((), ())),
                preferred_element_type=jnp.float32,
            ).astype(jnp.bfloat16)
            if i == 0:
                pl.semaphore_wait(barrier_sem, 1)
            rdma = pltpu.make_async_remote_copy(
                src_ref=accb_ref.at[sl],
                dst_ref=recv_ref.at[sl],
                send_sem=send_sems.at[i],
                recv_sem=recv_sems.at[i],
                device_id=nbr_y,
                device_id_type=pl.DeviceIdType.MESH,
            )
            rdma.start()
            rdmas.append(rdma)

        for i in range(C):
            sl = pl.ds(i * ck, ck)
            rdmas[i].wait_recv()
            out_ref[sl, :] = (accb_ref[sl, :].astype(jnp.float32)
                              + recv_ref[sl, :].astype(jnp.float32))

        for i in range(C):
            rdmas[i].wait_send()

    return pl.pallas_call(
        body,
        out_shape=jax.ShapeDtypeStruct((m, d), jnp.float32),
        in_specs=[
            pl.BlockSpec(memory_space=pltpu.VMEM),
            pl.BlockSpec(memory_space=pltpu.VMEM),
        ],
        out_specs=pl.BlockSpec(memory_space=pltpu.VMEM),
        scratch_shapes=[
            pltpu.VMEM((m, d), jnp.bfloat16),
            pltpu.VMEM((m, d), jnp.bfloat16),
            pltpu.SemaphoreType.DMA((C,)),
            pltpu.SemaphoreType.DMA((C,)),
        ],
        compiler_params=pltpu.CompilerParams(collective_id=0),
    )(dy, W)
